# baseline (speedup 1.0000x reference)
import functools

import jax
import jax.numpy as jnp
from jax.experimental import pallas as pl
from jax.experimental.pallas import tpu as pltpu

_N_TOKENS = 16384
_IN_DIM = 2048
_HIDDEN = 64
_OUT_DIM = 16
_BLOCK = 512


def _router_body(x_ref, w1_ref, w2_ref, w3_ref,
                 gates_ref, load_ref, idx_ref):
    i = pl.program_id(0)

    h = jnp.maximum(
        jnp.dot(x_ref[...], w1_ref[...], preferred_element_type=jnp.float32),
        0.0)
    h = jnp.maximum(
        jnp.dot(h, w2_ref[...], preferred_element_type=jnp.float32), 0.0)
    logits = jnp.dot(h, w3_ref[...], preferred_element_type=jnp.float32)

    lane_f = jax.lax.broadcasted_iota(jnp.int32, logits.shape, 1).astype(
        jnp.float32)

    v1 = jnp.max(logits, axis=1, keepdims=True)
    big = jnp.float32(_OUT_DIM)
    idx1_f = jnp.min(jnp.where(logits == v1, lane_f, big), axis=1,
                     keepdims=True)
    mask1 = lane_f == idx1_f

    neg = jnp.float32(-jnp.inf)
    l2 = jnp.where(mask1, neg, logits)
    v2 = jnp.max(l2, axis=1, keepdims=True)
    idx2_f = jnp.min(jnp.where(l2 == v2, lane_f, big), axis=1, keepdims=True)
    mask2 = lane_f == idx2_f

    e2 = jnp.exp(v2 - v1)
    denom = 1.0 + e2
    g1 = 1.0 / denom
    g2 = e2 / denom

    gates = jnp.where(mask1, g1, jnp.where(mask2, g2, 0.0))
    gates_ref[...] = gates
    idx_ref[...] = jnp.concatenate(
        [idx1_f.astype(jnp.int32), idx2_f.astype(jnp.int32)], axis=1)

    partial = jnp.sum((gates > 0.0).astype(jnp.int32), axis=0, keepdims=True)

    @pl.when(i == 0)
    def _init():
        load_ref[...] = jnp.zeros_like(load_ref)

    load_ref[...] += partial


@functools.partial(jax.jit, static_argnames=())
def kernel(x, We1, We2, We3, Wn1, Wn2, Wn3):
    del Wn1, Wn2, Wn3
    n_blocks = _N_TOKENS // _BLOCK
    gates, load, idx = pl.pallas_call(
        _router_body,
        grid=(n_blocks,),
        in_specs=[
            pl.BlockSpec((_BLOCK, _IN_DIM), lambda i: (i, 0)),
            pl.BlockSpec((_IN_DIM, _HIDDEN), lambda i: (0, 0)),
            pl.BlockSpec((_HIDDEN, _HIDDEN), lambda i: (0, 0)),
            pl.BlockSpec((_HIDDEN, _OUT_DIM), lambda i: (0, 0)),
        ],
        out_specs=[
            pl.BlockSpec((_BLOCK, _OUT_DIM), lambda i: (i, 0)),
            pl.BlockSpec((1, _OUT_DIM), lambda i: (0, 0)),
            pl.BlockSpec((_BLOCK, 2), lambda i: (i, 0)),
        ],
        out_shape=[
            jax.ShapeDtypeStruct((_N_TOKENS, _OUT_DIM), jnp.float32),
            jax.ShapeDtypeStruct((1, _OUT_DIM), jnp.int32),
            jax.ShapeDtypeStruct((_N_TOKENS, 2), jnp.int32),
        ],
        compiler_params=pltpu.CompilerParams(
            dimension_semantics=("arbitrary",),
        ),
    )(x, We1, We2, We3)
    return gates, load.reshape(_OUT_DIM), idx


# speedup vs baseline: 1.2196x; 1.2196x over previous
import functools

import jax
import jax.numpy as jnp
from jax.experimental import pallas as pl
from jax.experimental.pallas import tpu as pltpu

_N_TOKENS = 16384
_IN_DIM = 2048
_HIDDEN = 64
_OUT_DIM = 16
_BLOCK = 2048


def _router_body(x_ref, w1_ref, w2_ref, w3_ref,
                 gates_ref, load_ref, idx_ref):
    i = pl.program_id(0)

    h = jnp.maximum(
        jnp.dot(x_ref[...], w1_ref[...], preferred_element_type=jnp.float32),
        0.0)
    h = jnp.maximum(
        jnp.dot(h, w2_ref[...], preferred_element_type=jnp.float32), 0.0)
    logits = jnp.dot(h, w3_ref[...], preferred_element_type=jnp.float32)

    lane_f = jax.lax.broadcasted_iota(jnp.int32, logits.shape, 1).astype(
        jnp.float32)

    v1 = jnp.max(logits, axis=1, keepdims=True)
    big = jnp.float32(_OUT_DIM)
    idx1_f = jnp.min(jnp.where(logits == v1, lane_f, big), axis=1,
                     keepdims=True)
    mask1 = lane_f == idx1_f

    neg = jnp.float32(-jnp.inf)
    l2 = jnp.where(mask1, neg, logits)
    v2 = jnp.max(l2, axis=1, keepdims=True)
    idx2_f = jnp.min(jnp.where(l2 == v2, lane_f, big), axis=1, keepdims=True)
    mask2 = lane_f == idx2_f

    e2 = jnp.exp(v2 - v1)
    denom = 1.0 + e2
    g1 = 1.0 / denom
    g2 = e2 / denom

    gates = jnp.where(mask1, g1, jnp.where(mask2, g2, 0.0))
    gates_ref[...] = gates
    idx_ref[...] = jnp.concatenate(
        [idx1_f.astype(jnp.int32), idx2_f.astype(jnp.int32)], axis=1)

    partial = jnp.sum((gates > 0.0).astype(jnp.int32), axis=0, keepdims=True)

    @pl.when(i == 0)
    def _init():
        load_ref[...] = jnp.zeros_like(load_ref)

    load_ref[...] += partial


@functools.partial(jax.jit, static_argnames=())
def kernel(x, We1, We2, We3, Wn1, Wn2, Wn3):
    del Wn1, Wn2, Wn3
    n_blocks = _N_TOKENS // _BLOCK
    gates, load, idx = pl.pallas_call(
        _router_body,
        grid=(n_blocks,),
        in_specs=[
            pl.BlockSpec((_BLOCK, _IN_DIM), lambda i: (i, 0)),
            pl.BlockSpec((_IN_DIM, _HIDDEN), lambda i: (0, 0)),
            pl.BlockSpec((_HIDDEN, _HIDDEN), lambda i: (0, 0)),
            pl.BlockSpec((_HIDDEN, _OUT_DIM), lambda i: (0, 0)),
        ],
        out_specs=[
            pl.BlockSpec((_BLOCK, _OUT_DIM), lambda i: (i, 0)),
            pl.BlockSpec((1, _OUT_DIM), lambda i: (0, 0)),
            pl.BlockSpec((_BLOCK, 2), lambda i: (i, 0)),
        ],
        out_shape=[
            jax.ShapeDtypeStruct((_N_TOKENS, _OUT_DIM), jnp.float32),
            jax.ShapeDtypeStruct((1, _OUT_DIM), jnp.int32),
            jax.ShapeDtypeStruct((_N_TOKENS, 2), jnp.int32),
        ],
        compiler_params=pltpu.CompilerParams(
            dimension_semantics=("arbitrary",),
        ),
    )(x, We1, We2, We3)
    return gates, load.reshape(_OUT_DIM), idx
